# baseline (speedup 1.0000x reference)
import functools

import jax
import jax.numpy as jnp
from jax import lax
from jax.experimental import pallas as pl
from jax.experimental.pallas import tpu as pltpu
from jax.experimental.pallas import tpu_sc as plsc

_INTERP = False

XPAD = 16


def _silu(v):
    return v * jax.nn.sigmoid(v)


def _proj_body(h_ref, x_ref, wat_ref, wbt_ref, p_ref, q_ref):
    hblk = h_ref[...]
    hdim = wat_ref.shape[1]
    xblk = jnp.pad(x_ref[...], ((0, 0), (0, hdim - x_ref.shape[1])))
    p_ref[:, :hdim] = jnp.dot(hblk, wat_ref[...], preferred_element_type=jnp.float32)
    p_ref[:, hdim:] = xblk
    q_ref[:, :hdim] = jnp.dot(hblk, wbt_ref[...], preferred_element_type=jnp.float32)
    q_ref[:, hdim:] = xblk


def _node_proj(h, x, wat, wbt):
    n, d = h.shape
    hdim = wat.shape[1]
    bn = 1000
    grid = (n // bn,)
    return pl.pallas_call(
        _proj_body,
        grid=grid,
        in_specs=[
            pl.BlockSpec((bn, d), lambda i: (i, 0)),
            pl.BlockSpec((bn, 3), lambda i: (i, 0)),
            pl.BlockSpec((d, hdim), lambda i: (0, 0)),
            pl.BlockSpec((d, hdim), lambda i: (0, 0)),
        ],
        out_specs=[
            pl.BlockSpec((bn, 2 * hdim), lambda i: (i, 0)),
            pl.BlockSpec((bn, 2 * hdim), lambda i: (i, 0)),
        ],
        out_shape=[
            jax.ShapeDtypeStruct((n, 2 * hdim), jnp.float32),
            jax.ShapeDtypeStruct((n, 2 * hdim), jnp.float32),
        ],
        interpret=_INTERP,
    )(h, x, wat, wbt)


def _edge_body(pq_ref, ea_ref, wd16_ref, wet_ref,
               b1_ref, w2t_ref, b2_ref, cw1t_ref, cb1_ref, cw2_ref,
               mout_ref, tout_ref):
    hdim = wd16_ref.shape[1]
    diff = pq_ref[:, hdim:hdim + XPAD]
    m1 = (pq_ref[:, :hdim]
          + jnp.dot(diff * diff, wd16_ref[...], preferred_element_type=jnp.float32)
          + jnp.dot(ea_ref[...], wet_ref[...], preferred_element_type=jnp.float32)
          + b1_ref[...])
    m = _silu(m1)
    bf = jnp.bfloat16
    m_ij = _silu(jnp.dot(m.astype(bf), w2t_ref[...].astype(bf),
                         preferred_element_type=jnp.float32) + b2_ref[...])
    c = _silu(jnp.dot(m_ij.astype(bf), cw1t_ref[...].astype(bf),
                      preferred_element_type=jnp.float32) + cb1_ref[...])
    coef = jnp.sum(c * cw2_ref[...], axis=1, keepdims=True)
    mout_ref[...] = m_ij
    tout_ref[...] = jnp.pad(diff * coef, ((0, 0), (0, hdim - diff.shape[1])))


def _edge_mlp(pq_rows, ea, wd16, wet, b1, w2t, b2, cw1t, cb1, cw2):
    e, w2 = pq_rows.shape
    hdim = w2 // 2
    de = ea.shape[1]
    be = 2000
    grid = (e // be,)
    full = lambda i: (0, 0)
    return pl.pallas_call(
        _edge_body,
        grid=grid,
        in_specs=[
            pl.BlockSpec((be, w2), lambda i: (i, 0)),
            pl.BlockSpec((be, de), lambda i: (i, 0)),
            pl.BlockSpec((XPAD, hdim), full),
            pl.BlockSpec((de, hdim), full),
            pl.BlockSpec((1, hdim), full),
            pl.BlockSpec((hdim, hdim), full),
            pl.BlockSpec((1, hdim), full),
            pl.BlockSpec((hdim, hdim), full),
            pl.BlockSpec((1, hdim), full),
            pl.BlockSpec((1, hdim), full),
        ],
        out_specs=[
            pl.BlockSpec((be, hdim), lambda i: (i, 0)),
            pl.BlockSpec((be, hdim), lambda i: (i, 0)),
        ],
        out_shape=[
            jax.ShapeDtypeStruct((e, hdim), jnp.float32),
            jax.ShapeDtypeStruct((e, hdim), jnp.float32),
        ],
        interpret=_INTERP,
    )(pq_rows, ea, wd16, wet, b1, w2t, b2, cw1t, cb1, cw2)


def _node_body(h_ref, x_ref, macc_ref, tacc_ref,
               w1at_ref, w1bt_ref, nb1_ref,
               w2t_ref, nb2_ref, g_ref, b_ref, ho_ref, xo_ref):
    m_i = macc_ref[...]
    x_agg = tacc_ref[...][:, :3]
    xo_ref[...] = x_ref[...] + x_agg
    hblk = h_ref[...]
    hh = _silu(jnp.dot(hblk, w1at_ref[...], preferred_element_type=jnp.float32)
               + jnp.dot(m_i, w1bt_ref[...], preferred_element_type=jnp.float32)
               + nb1_ref[...])
    ho = hblk + jnp.dot(hh, w2t_ref[...], preferred_element_type=jnp.float32) + nb2_ref[...]
    mu = jnp.mean(ho, axis=1, keepdims=True)
    ctr = ho - mu
    var = jnp.mean(ctr * ctr, axis=1, keepdims=True)
    ho_ref[...] = ctr * lax.rsqrt(var + 1e-5) * g_ref[...] + b_ref[...]


def _node_update(h, x, macc, tacc, w1at, w1bt, nb1, w2t, nb2, g, b):
    n, d = h.shape
    hdim = w1bt.shape[0]
    bn = 1000
    grid = (n // bn,)
    full = lambda i: (0, 0)
    return pl.pallas_call(
        _node_body,
        grid=grid,
        in_specs=[
            pl.BlockSpec((bn, d), lambda i: (i, 0)),
            pl.BlockSpec((bn, 3), lambda i: (i, 0)),
            pl.BlockSpec((bn, hdim), lambda i: (i, 0)),
            pl.BlockSpec((bn, hdim), lambda i: (i, 0)),
            pl.BlockSpec((d, hdim), full),
            pl.BlockSpec((hdim, hdim), full),
            pl.BlockSpec((1, hdim), full),
            pl.BlockSpec((hdim, d), full),
            pl.BlockSpec((1, d), full),
            pl.BlockSpec((1, d), full),
            pl.BlockSpec((1, d), full),
        ],
        out_specs=[
            pl.BlockSpec((bn, d), lambda i: (i, 0)),
            pl.BlockSpec((bn, 3), lambda i: (i, 0)),
        ],
        out_shape=[
            jax.ShapeDtypeStruct((n, d), jnp.float32),
            jax.ShapeDtypeStruct((n, 3), jnp.float32),
        ],
        interpret=_INTERP,
    )(h, x, macc, tacc, w1at, w1bt, nb1, w2t, nb2, g, b)


def _edge_gather(p, q, row, col):
    e = row.shape[0]
    w2 = p.shape[1]
    ch = 80
    nw = 32
    ipe = e // nw
    ipw = ipe // ch
    hdim = w2 // 2
    mesh = plsc.VectorSubcoreMesh(core_axis_name="c", subcore_axis_name="s")

    @functools.partial(
        pl.kernel,
        out_type=jax.ShapeDtypeStruct((e, w2), jnp.float32),
        mesh=mesh,
        scratch_types=[
            pltpu.VMEM((ipe,), jnp.int32),
            pltpu.VMEM((ipe,), jnp.int32),
            pltpu.VMEM((ch, w2), jnp.float32),
            pltpu.VMEM((ch, w2), jnp.float32),
            pltpu.VMEM((ch, w2), jnp.float32),
            pltpu.VMEM((ch, w2), jnp.float32),
            pltpu.SemaphoreType.DMA,
            pltpu.SemaphoreType.DMA,
            pltpu.SemaphoreType.DMA,
            pltpu.SemaphoreType.DMA,
            pltpu.SemaphoreType.DMA,
            pltpu.SemaphoreType.DMA,
        ],
    )
    def gather_kernel(p_hbm, q_hbm, row_hbm, col_hbm, po,
                      ir_all, ic_all, pv0, qv0, pv1, qv1,
                      sp0, sq0, sp1, sq1, sw0, sw1):
        cid = lax.axis_index("c")
        sid = lax.axis_index("s")
        wid = sid * 2 + cid
        e0 = wid * ipe
        pltpu.sync_copy(row_hbm.at[pl.ds(e0, ipe)], ir_all)
        pltpu.sync_copy(col_hbm.at[pl.ds(e0, ipe)], ic_all)
        pvs = (pv0, pv1)
        qvs = (qv0, qv1)
        sps = (sp0, sp1)
        sqs = (sq0, sq1)
        sws = (sw0, sw1)

        def issue(j, s):
            lb = j * ch
            pltpu.async_copy(p_hbm.at[ir_all.at[pl.ds(lb, ch)]], pvs[s], sps[s])
            pltpu.async_copy(q_hbm.at[ic_all.at[pl.ds(lb, ch)]], qvs[s], sqs[s])

        issue(0, 0)

        def pair(i, carry):
            for b in (0, 1):
                j = 2 * i + b
                jn = j + 1

                @pl.when(jn < ipw)
                def _():
                    @pl.when(j >= 1)
                    def _():
                        pltpu.make_async_copy(
                            pvs[1 - b], po.at[pl.ds(e0 + (j - 1) * ch, ch)],
                            sws[1 - b]).wait()

                    issue(jn, 1 - b)

                @pl.when(j < ipw)
                def _():
                    lb = j * ch
                    pltpu.make_async_copy(
                        p_hbm.at[ir_all.at[pl.ds(lb, ch)]], pvs[b], sps[b]).wait()
                    pltpu.make_async_copy(
                        q_hbm.at[ic_all.at[pl.ds(lb, ch)]], qvs[b], sqs[b]).wait()
                    p_v = pvs[b]
                    q_v = qvs[b]

                    @plsc.parallel_loop(0, ch, step=1, unroll=4)
                    def crow(r):
                        for k in range(hdim // 16):
                            sl = pl.ds(k * 16, 16)
                            p_v[r, sl] = p_v[r, sl] + q_v[r, sl]
                        dsl = pl.ds(hdim, 16)
                        p_v[r, dsl] = p_v[r, dsl] - q_v[r, dsl]

                    pltpu.async_copy(p_v, po.at[pl.ds(e0 + lb, ch)], sws[b])

            return carry

        lax.fori_loop(0, (ipw + 1) // 2, pair, 0)
        for jl in (ipw - 1, ipw - 2):
            pltpu.make_async_copy(
                pvs[jl % 2], po.at[pl.ds(e0 + jl * ch, ch)], sws[jl % 2]).wait()

    return gather_kernel(p, q, row, col)


def _edge_scatter(mupd, tupd, row, zm):
    e, hdim = mupd.shape
    n = zm.shape[0]
    ch = 80
    npc = 16
    ipe = e // npc
    ipw = ipe // ch
    rows_per_tile = n // 16
    stage_steps = rows_per_tile // ch
    mesh = plsc.VectorSubcoreMesh(core_axis_name="c", subcore_axis_name="s")

    @functools.partial(
        pl.kernel,
        out_type=[
            jax.ShapeDtypeStruct((n, hdim), jnp.float32),
            jax.ShapeDtypeStruct((n, hdim), jnp.float32),
        ],
        mesh=mesh,
        scratch_types=[
            pltpu.VMEM((ch,), jnp.int32),
            pltpu.VMEM((ch,), jnp.int32),
            pltpu.VMEM((ch, hdim), jnp.float32),
            pltpu.VMEM((ch, hdim), jnp.float32),
            pltpu.VMEM_SHARED((n, hdim), jnp.float32),
            pltpu.SemaphoreType.DMA,
            pltpu.SemaphoreType.DMA,
            pltpu.SemaphoreType.DMA,
            pltpu.SemaphoreType.DMA,
        ],
    )
    def scatter_kernel(m_hbm, t_hbm, row_hbm, zm_hbm, macc, tacc,
                       iv0, iv1, uv0, uv1, acc_sh, su0, su1, si0, si1):
        cid = lax.axis_index("c")
        sid = lax.axis_index("s")
        r0 = sid * rows_per_tile
        e0 = sid * ipe
        ivs = (iv0, iv1)
        uvs = (uv0, uv1)
        sus = (su0, su1)
        sis = (si0, si1)

        def zstep(j, carry):
            rb = r0 + j * ch
            pltpu.sync_copy(zm_hbm.at[pl.ds(rb, ch)], uv0)
            pltpu.sync_copy(uv0, acc_sh.at[pl.ds(rb, ch)])
            return carry

        lax.fori_loop(0, stage_steps, zstep, 0)
        plsc.subcore_barrier()

        def make_loop(src_hbm):
            def issue(j, s):
                base = e0 + j * ch
                pltpu.async_copy(row_hbm.at[pl.ds(base, ch)], ivs[s], sis[s])
                pltpu.async_copy(src_hbm.at[pl.ds(base, ch)], uvs[s], sus[s])

            issue(0, 0)

            def pair(i, carry):
                for b in (0, 1):
                    j = 2 * i + b
                    jn = j + 1

                    @pl.when(jn < ipw)
                    def _():
                        issue(jn, 1 - b)

                    @pl.when(j < ipw)
                    def _():
                        base = e0 + j * ch
                        pltpu.make_async_copy(
                            row_hbm.at[pl.ds(base, ch)], ivs[b], sis[b]).wait()
                        pltpu.make_async_copy(
                            src_hbm.at[pl.ds(base, ch)], uvs[b], sus[b]).wait()
                        pltpu.sync_copy(uvs[b], acc_sh.at[ivs[b]], add=True)

                return carry

            lax.fori_loop(0, (ipw + 1) // 2, pair, 0)

        @pl.when(cid == 0)
        def _():
            make_loop(m_hbm)

        @pl.when(cid == 1)
        def _():
            make_loop(t_hbm)

        plsc.subcore_barrier()

        def dstep(j, carry):
            rb = r0 + j * ch
            pltpu.sync_copy(acc_sh.at[pl.ds(rb, ch)], uv0)

            @pl.when(cid == 0)
            def _():
                pltpu.sync_copy(uv0, macc.at[pl.ds(rb, ch)])

            @pl.when(cid == 1)
            def _():
                pltpu.sync_copy(uv0, tacc.at[pl.ds(rb, ch)])

            return carry

        lax.fori_loop(0, stage_steps, dstep, 0)

    return scatter_kernel(mupd, tupd, row, zm)


def kernel(h, x, edge_index, edge_attr, msg_w1, msg_b1, msg_w2, msg_b2,
           node_w1, node_b1, node_w2, node_b2, coord_w1, coord_b1, coord_w2,
           ln_g, ln_b):
    n, d = h.shape
    e = edge_index.shape[1]
    hdim = msg_w2.shape[0]
    de = edge_attr.shape[1]

    row = edge_index[0]
    col = edge_index[1]

    wat = msg_w1[:, :d].T
    wbt = msg_w1[:, d:2 * d].T
    wd = msg_w1[:, 2 * d]
    wet = msg_w1[:, 2 * d + 1:].T
    b1_eff = (msg_b1 + 1e-8 * wd)[None, :]
    wd16 = jnp.broadcast_to(wd[None, :], (XPAD, hdim))
    w2t = msg_w2.T
    b2 = msg_b2[None, :]
    cw1t = coord_w1.T
    cb1 = coord_b1[None, :]
    cw2 = coord_w2
    w1at = node_w1[:, :d].T
    w1bt = node_w1[:, d:].T
    nb1 = node_b1[None, :]
    nw2t = node_w2.T
    nb2 = node_b2[None, :]
    g = ln_g[None, :]
    b = ln_b[None, :]

    p, q = _node_proj(h, x, wat, wbt)

    pq_rows = _edge_gather(p, q, row, col)

    mout, tout = _edge_mlp(pq_rows, edge_attr,
                           wd16, wet, b1_eff, w2t, b2, cw1t, cb1, cw2)

    n_pad = -(-n // 2048) * 2048
    zm = jnp.zeros((n_pad, hdim), dtype=jnp.float32)
    macc, tacc = _edge_scatter(mout, tout, row, zm)

    h_out, x_out = _node_update(h, x, macc, tacc,
                                w1at, w1bt, nb1, nw2t, nb2, g, b)
    return (h_out, x_out)


# speedup vs baseline: 1.0020x; 1.0020x over previous
import functools

import jax
import jax.numpy as jnp
from jax import lax
from jax.experimental import pallas as pl
from jax.experimental.pallas import tpu as pltpu
from jax.experimental.pallas import tpu_sc as plsc

_INTERP = False

XPAD = 16


def _silu(v):
    return v * jax.nn.sigmoid(v)


def _proj_body(h_ref, x_ref, wat_ref, wbt_ref, p_ref, q_ref):
    hblk = h_ref[...]
    hdim = wat_ref.shape[1]
    xblk = jnp.pad(x_ref[...], ((0, 0), (0, hdim - x_ref.shape[1])))
    p_ref[:, :hdim] = jnp.dot(hblk, wat_ref[...], preferred_element_type=jnp.float32)
    p_ref[:, hdim:] = xblk
    q_ref[:, :hdim] = jnp.dot(hblk, wbt_ref[...], preferred_element_type=jnp.float32)
    q_ref[:, hdim:] = xblk


def _node_proj(h, x, wat, wbt):
    n, d = h.shape
    hdim = wat.shape[1]
    bn = 1000
    grid = (n // bn,)
    return pl.pallas_call(
        _proj_body,
        grid=grid,
        in_specs=[
            pl.BlockSpec((bn, d), lambda i: (i, 0)),
            pl.BlockSpec((bn, 3), lambda i: (i, 0)),
            pl.BlockSpec((d, hdim), lambda i: (0, 0)),
            pl.BlockSpec((d, hdim), lambda i: (0, 0)),
        ],
        out_specs=[
            pl.BlockSpec((bn, 2 * hdim), lambda i: (i, 0)),
            pl.BlockSpec((bn, 2 * hdim), lambda i: (i, 0)),
        ],
        out_shape=[
            jax.ShapeDtypeStruct((n, 2 * hdim), jnp.float32),
            jax.ShapeDtypeStruct((n, 2 * hdim), jnp.float32),
        ],
        interpret=_INTERP,
    )(h, x, wat, wbt)


def _edge_body(pq_ref, ea_ref, wd16_ref, wet_ref,
               b1_ref, w2t_ref, b2_ref, cw1t_ref, cb1_ref, cw2_ref,
               mout_ref, tout_ref):
    hdim = wd16_ref.shape[1]
    diff = pq_ref[:, hdim:hdim + XPAD]
    m1 = (pq_ref[:, :hdim]
          + jnp.dot(diff * diff, wd16_ref[...], preferred_element_type=jnp.float32)
          + jnp.dot(ea_ref[...], wet_ref[...], preferred_element_type=jnp.float32)
          + b1_ref[...])
    m = _silu(m1)
    m_ij = _silu(jnp.dot(m, w2t_ref[...], preferred_element_type=jnp.float32)
                 + b2_ref[...])
    c = _silu(jnp.dot(m_ij, cw1t_ref[...], preferred_element_type=jnp.float32)
              + cb1_ref[...])
    coef = jnp.sum(c * cw2_ref[...], axis=1, keepdims=True)
    mout_ref[...] = m_ij
    tout_ref[...] = jnp.pad(diff * coef, ((0, 0), (0, hdim - diff.shape[1])))


def _edge_mlp(pq_rows, ea, wd16, wet, b1, w2t, b2, cw1t, cb1, cw2):
    e, w2 = pq_rows.shape
    hdim = w2 // 2
    de = ea.shape[1]
    be = 2000
    grid = (e // be,)
    full = lambda i: (0, 0)
    return pl.pallas_call(
        _edge_body,
        grid=grid,
        in_specs=[
            pl.BlockSpec((be, w2), lambda i: (i, 0)),
            pl.BlockSpec((be, de), lambda i: (i, 0)),
            pl.BlockSpec((XPAD, hdim), full),
            pl.BlockSpec((de, hdim), full),
            pl.BlockSpec((1, hdim), full),
            pl.BlockSpec((hdim, hdim), full),
            pl.BlockSpec((1, hdim), full),
            pl.BlockSpec((hdim, hdim), full),
            pl.BlockSpec((1, hdim), full),
            pl.BlockSpec((1, hdim), full),
        ],
        out_specs=[
            pl.BlockSpec((be, hdim), lambda i: (i, 0)),
            pl.BlockSpec((be, hdim), lambda i: (i, 0)),
        ],
        out_shape=[
            jax.ShapeDtypeStruct((e, hdim), jnp.float32),
            jax.ShapeDtypeStruct((e, hdim), jnp.float32),
        ],
        interpret=_INTERP,
    )(pq_rows, ea, wd16, wet, b1, w2t, b2, cw1t, cb1, cw2)


def _node_body(h_ref, x_ref, macc_ref, tacc_ref,
               w1at_ref, w1bt_ref, nb1_ref,
               w2t_ref, nb2_ref, g_ref, b_ref, ho_ref, xo_ref):
    m_i = macc_ref[...]
    x_agg = tacc_ref[...][:, :3]
    xo_ref[...] = x_ref[...] + x_agg
    hblk = h_ref[...]
    hh = _silu(jnp.dot(hblk, w1at_ref[...], preferred_element_type=jnp.float32)
               + jnp.dot(m_i, w1bt_ref[...], preferred_element_type=jnp.float32)
               + nb1_ref[...])
    ho = hblk + jnp.dot(hh, w2t_ref[...], preferred_element_type=jnp.float32) + nb2_ref[...]
    mu = jnp.mean(ho, axis=1, keepdims=True)
    ctr = ho - mu
    var = jnp.mean(ctr * ctr, axis=1, keepdims=True)
    ho_ref[...] = ctr * lax.rsqrt(var + 1e-5) * g_ref[...] + b_ref[...]


def _node_update(h, x, macc, tacc, w1at, w1bt, nb1, w2t, nb2, g, b):
    n, d = h.shape
    hdim = w1bt.shape[0]
    bn = 1000
    grid = (n // bn,)
    full = lambda i: (0, 0)
    return pl.pallas_call(
        _node_body,
        grid=grid,
        in_specs=[
            pl.BlockSpec((bn, d), lambda i: (i, 0)),
            pl.BlockSpec((bn, 3), lambda i: (i, 0)),
            pl.BlockSpec((bn, hdim), lambda i: (i, 0)),
            pl.BlockSpec((bn, hdim), lambda i: (i, 0)),
            pl.BlockSpec((d, hdim), full),
            pl.BlockSpec((hdim, hdim), full),
            pl.BlockSpec((1, hdim), full),
            pl.BlockSpec((hdim, d), full),
            pl.BlockSpec((1, d), full),
            pl.BlockSpec((1, d), full),
            pl.BlockSpec((1, d), full),
        ],
        out_specs=[
            pl.BlockSpec((bn, d), lambda i: (i, 0)),
            pl.BlockSpec((bn, 3), lambda i: (i, 0)),
        ],
        out_shape=[
            jax.ShapeDtypeStruct((n, d), jnp.float32),
            jax.ShapeDtypeStruct((n, 3), jnp.float32),
        ],
        interpret=_INTERP,
    )(h, x, macc, tacc, w1at, w1bt, nb1, w2t, nb2, g, b)


def _edge_gather(p, q, row, col):
    e = row.shape[0]
    w2 = p.shape[1]
    ch = 80
    nw = 32
    ipe = e // nw
    ipw = ipe // ch
    hdim = w2 // 2
    mesh = plsc.VectorSubcoreMesh(core_axis_name="c", subcore_axis_name="s")

    @functools.partial(
        pl.kernel,
        out_type=jax.ShapeDtypeStruct((e, w2), jnp.float32),
        mesh=mesh,
        scratch_types=[
            pltpu.VMEM((ipe,), jnp.int32),
            pltpu.VMEM((ipe,), jnp.int32),
            pltpu.VMEM((ch, w2), jnp.float32),
            pltpu.VMEM((ch, w2), jnp.float32),
            pltpu.VMEM((ch, w2), jnp.float32),
            pltpu.VMEM((ch, w2), jnp.float32),
            pltpu.SemaphoreType.DMA,
            pltpu.SemaphoreType.DMA,
            pltpu.SemaphoreType.DMA,
            pltpu.SemaphoreType.DMA,
            pltpu.SemaphoreType.DMA,
            pltpu.SemaphoreType.DMA,
        ],
    )
    def gather_kernel(p_hbm, q_hbm, row_hbm, col_hbm, po,
                      ir_all, ic_all, pv0, qv0, pv1, qv1,
                      sp0, sq0, sp1, sq1, sw0, sw1):
        cid = lax.axis_index("c")
        sid = lax.axis_index("s")
        wid = sid * 2 + cid
        e0 = wid * ipe
        pltpu.sync_copy(row_hbm.at[pl.ds(e0, ipe)], ir_all)
        pltpu.sync_copy(col_hbm.at[pl.ds(e0, ipe)], ic_all)
        pvs = (pv0, pv1)
        qvs = (qv0, qv1)
        sps = (sp0, sp1)
        sqs = (sq0, sq1)
        sws = (sw0, sw1)

        def issue(j, s):
            lb = j * ch
            pltpu.async_copy(p_hbm.at[ir_all.at[pl.ds(lb, ch)]], pvs[s], sps[s])
            pltpu.async_copy(q_hbm.at[ic_all.at[pl.ds(lb, ch)]], qvs[s], sqs[s])

        issue(0, 0)

        def pair(i, carry):
            for b in (0, 1):
                j = 2 * i + b
                jn = j + 1

                @pl.when(jn < ipw)
                def _():
                    @pl.when(j >= 1)
                    def _():
                        pltpu.make_async_copy(
                            pvs[1 - b], po.at[pl.ds(e0 + (j - 1) * ch, ch)],
                            sws[1 - b]).wait()

                    issue(jn, 1 - b)

                @pl.when(j < ipw)
                def _():
                    lb = j * ch
                    pltpu.make_async_copy(
                        p_hbm.at[ir_all.at[pl.ds(lb, ch)]], pvs[b], sps[b]).wait()
                    pltpu.make_async_copy(
                        q_hbm.at[ic_all.at[pl.ds(lb, ch)]], qvs[b], sqs[b]).wait()
                    p_v = pvs[b]
                    q_v = qvs[b]

                    @plsc.parallel_loop(0, ch, step=1, unroll=4)
                    def crow(r):
                        for k in range(hdim // 16):
                            sl = pl.ds(k * 16, 16)
                            p_v[r, sl] = p_v[r, sl] + q_v[r, sl]
                        dsl = pl.ds(hdim, 16)
                        p_v[r, dsl] = p_v[r, dsl] - q_v[r, dsl]

                    pltpu.async_copy(p_v, po.at[pl.ds(e0 + lb, ch)], sws[b])

            return carry

        lax.fori_loop(0, (ipw + 1) // 2, pair, 0)
        for jl in (ipw - 1, ipw - 2):
            pltpu.make_async_copy(
                pvs[jl % 2], po.at[pl.ds(e0 + jl * ch, ch)], sws[jl % 2]).wait()

    return gather_kernel(p, q, row, col)


def _edge_scatter(mupd, tupd, row, zm):
    e, hdim = mupd.shape
    n = zm.shape[0]
    ch = 80
    npc = 16
    ipe = e // npc
    ipw = ipe // ch
    rows_per_tile = n // 16
    stage_steps = rows_per_tile // ch
    mesh = plsc.VectorSubcoreMesh(core_axis_name="c", subcore_axis_name="s")

    @functools.partial(
        pl.kernel,
        out_type=[
            jax.ShapeDtypeStruct((n, hdim), jnp.float32),
            jax.ShapeDtypeStruct((n, hdim), jnp.float32),
        ],
        mesh=mesh,
        scratch_types=[
            pltpu.VMEM((ch,), jnp.int32),
            pltpu.VMEM((ch,), jnp.int32),
            pltpu.VMEM((ch, hdim), jnp.float32),
            pltpu.VMEM((ch, hdim), jnp.float32),
            pltpu.VMEM_SHARED((n, hdim), jnp.float32),
            pltpu.SemaphoreType.DMA,
            pltpu.SemaphoreType.DMA,
            pltpu.SemaphoreType.DMA,
            pltpu.SemaphoreType.DMA,
        ],
    )
    def scatter_kernel(m_hbm, t_hbm, row_hbm, zm_hbm, macc, tacc,
                       iv0, iv1, uv0, uv1, acc_sh, su0, su1, si0, si1):
        cid = lax.axis_index("c")
        sid = lax.axis_index("s")
        r0 = sid * rows_per_tile
        e0 = sid * ipe
        ivs = (iv0, iv1)
        uvs = (uv0, uv1)
        sus = (su0, su1)
        sis = (si0, si1)

        def zstep(j, carry):
            rb = r0 + j * ch
            pltpu.sync_copy(zm_hbm.at[pl.ds(rb, ch)], uv0)
            pltpu.sync_copy(uv0, acc_sh.at[pl.ds(rb, ch)])
            return carry

        lax.fori_loop(0, stage_steps, zstep, 0)
        plsc.subcore_barrier()

        def make_loop(src_hbm):
            def issue(j, s):
                base = e0 + j * ch
                pltpu.async_copy(row_hbm.at[pl.ds(base, ch)], ivs[s], sis[s])
                pltpu.async_copy(src_hbm.at[pl.ds(base, ch)], uvs[s], sus[s])

            issue(0, 0)

            def pair(i, carry):
                for b in (0, 1):
                    j = 2 * i + b
                    jn = j + 1

                    @pl.when(jn < ipw)
                    def _():
                        issue(jn, 1 - b)

                    @pl.when(j < ipw)
                    def _():
                        base = e0 + j * ch
                        pltpu.make_async_copy(
                            row_hbm.at[pl.ds(base, ch)], ivs[b], sis[b]).wait()
                        pltpu.make_async_copy(
                            src_hbm.at[pl.ds(base, ch)], uvs[b], sus[b]).wait()
                        pltpu.sync_copy(uvs[b], acc_sh.at[ivs[b]], add=True)

                return carry

            lax.fori_loop(0, (ipw + 1) // 2, pair, 0)

        @pl.when(cid == 0)
        def _():
            make_loop(m_hbm)

        @pl.when(cid == 1)
        def _():
            make_loop(t_hbm)

        plsc.subcore_barrier()

        def dstep(j, carry):
            rb = r0 + j * ch
            pltpu.sync_copy(acc_sh.at[pl.ds(rb, ch)], uv0)

            @pl.when(cid == 0)
            def _():
                pltpu.sync_copy(uv0, macc.at[pl.ds(rb, ch)])

            @pl.when(cid == 1)
            def _():
                pltpu.sync_copy(uv0, tacc.at[pl.ds(rb, ch)])

            return carry

        lax.fori_loop(0, stage_steps, dstep, 0)

    return scatter_kernel(mupd, tupd, row, zm)


def kernel(h, x, edge_index, edge_attr, msg_w1, msg_b1, msg_w2, msg_b2,
           node_w1, node_b1, node_w2, node_b2, coord_w1, coord_b1, coord_w2,
           ln_g, ln_b):
    n, d = h.shape
    e = edge_index.shape[1]
    hdim = msg_w2.shape[0]
    de = edge_attr.shape[1]

    row = edge_index[0]
    col = edge_index[1]

    wat = msg_w1[:, :d].T
    wbt = msg_w1[:, d:2 * d].T
    wd = msg_w1[:, 2 * d]
    wet = msg_w1[:, 2 * d + 1:].T
    b1_eff = (msg_b1 + 1e-8 * wd)[None, :]
    wd16 = jnp.broadcast_to(wd[None, :], (XPAD, hdim))
    w2t = msg_w2.T
    b2 = msg_b2[None, :]
    cw1t = coord_w1.T
    cb1 = coord_b1[None, :]
    cw2 = coord_w2
    w1at = node_w1[:, :d].T
    w1bt = node_w1[:, d:].T
    nb1 = node_b1[None, :]
    nw2t = node_w2.T
    nb2 = node_b2[None, :]
    g = ln_g[None, :]
    b = ln_b[None, :]

    p, q = _node_proj(h, x, wat, wbt)

    pq_rows = _edge_gather(p, q, row, col)

    mout, tout = _edge_mlp(pq_rows, edge_attr,
                           wd16, wet, b1_eff, w2t, b2, cw1t, cb1, cw2)

    n_pad = -(-n // 2048) * 2048
    zm = jnp.zeros((n_pad, hdim), dtype=jnp.float32)
    macc, tacc = _edge_scatter(mout, tout, row, zm)

    h_out, x_out = _node_update(h, x, macc, tacc,
                                w1at, w1bt, nb1, nw2t, nb2, g, b)
    return (h_out, x_out)


# speedup vs baseline: 1.0744x; 1.0722x over previous
import functools

import jax
import jax.numpy as jnp
from jax import lax
from jax.experimental import pallas as pl
from jax.experimental.pallas import tpu as pltpu
from jax.experimental.pallas import tpu_sc as plsc

_INTERP = False

XPAD = 16


def _silu(v):
    return v * jax.nn.sigmoid(v)


def _proj_body(h_ref, x_ref, wat_ref, wbt_ref, p_ref, q_ref):
    hblk = h_ref[...]
    hdim = wat_ref.shape[1]
    xblk = jnp.pad(x_ref[...], ((0, 0), (0, hdim - x_ref.shape[1])))
    p_ref[:, :hdim] = jnp.dot(hblk, wat_ref[...], preferred_element_type=jnp.float32)
    p_ref[:, hdim:] = xblk
    q_ref[:, :hdim] = jnp.dot(hblk, wbt_ref[...], preferred_element_type=jnp.float32)
    q_ref[:, hdim:] = xblk


def _node_proj(h, x, wat, wbt):
    n, d = h.shape
    hdim = wat.shape[1]
    bn = 1000
    grid = (n // bn,)
    return pl.pallas_call(
        _proj_body,
        grid=grid,
        in_specs=[
            pl.BlockSpec((bn, d), lambda i: (i, 0)),
            pl.BlockSpec((bn, 3), lambda i: (i, 0)),
            pl.BlockSpec((d, hdim), lambda i: (0, 0)),
            pl.BlockSpec((d, hdim), lambda i: (0, 0)),
        ],
        out_specs=[
            pl.BlockSpec((bn, 2 * hdim), lambda i: (i, 0)),
            pl.BlockSpec((bn, 2 * hdim), lambda i: (i, 0)),
        ],
        out_shape=[
            jax.ShapeDtypeStruct((n, 2 * hdim), jnp.float32),
            jax.ShapeDtypeStruct((n, 2 * hdim), jnp.float32),
        ],
        interpret=_INTERP,
    )(h, x, wat, wbt)


def _edge_body(pq_ref, ea_ref, wd16_ref, wet_ref,
               b1_ref, w2t_ref, b2_ref, cw1t_ref, cb1_ref, cw2_ref,
               mout_ref, tout_ref):
    hdim = wd16_ref.shape[1]
    diff = pq_ref[:, hdim:hdim + XPAD]
    m1 = (pq_ref[:, :hdim]
          + jnp.dot(diff * diff, wd16_ref[...], preferred_element_type=jnp.float32)
          + jnp.dot(ea_ref[...], wet_ref[...], preferred_element_type=jnp.float32)
          + b1_ref[...])
    m = _silu(m1)
    m_ij = _silu(jnp.dot(m, w2t_ref[...], preferred_element_type=jnp.float32)
                 + b2_ref[...])
    c = _silu(jnp.dot(m_ij, cw1t_ref[...], preferred_element_type=jnp.float32)
              + cb1_ref[...])
    coef = jnp.sum(c * cw2_ref[...], axis=1, keepdims=True)
    mout_ref[...] = m_ij
    tout_ref[...] = jnp.pad(diff * coef, ((0, 0), (0, hdim - diff.shape[1])))


def _edge_mlp(pq_rows, ea, wd16, wet, b1, w2t, b2, cw1t, cb1, cw2):
    e, w2 = pq_rows.shape
    hdim = w2 // 2
    de = ea.shape[1]
    be = 2000
    grid = (e // be,)
    full = lambda i: (0, 0)
    return pl.pallas_call(
        _edge_body,
        grid=grid,
        in_specs=[
            pl.BlockSpec((be, w2), lambda i: (i, 0)),
            pl.BlockSpec((be, de), lambda i: (i, 0)),
            pl.BlockSpec((XPAD, hdim), full),
            pl.BlockSpec((de, hdim), full),
            pl.BlockSpec((1, hdim), full),
            pl.BlockSpec((hdim, hdim), full),
            pl.BlockSpec((1, hdim), full),
            pl.BlockSpec((hdim, hdim), full),
            pl.BlockSpec((1, hdim), full),
            pl.BlockSpec((1, hdim), full),
        ],
        out_specs=[
            pl.BlockSpec((be, hdim), lambda i: (i, 0)),
            pl.BlockSpec((be, hdim), lambda i: (i, 0)),
        ],
        out_shape=[
            jax.ShapeDtypeStruct((e, hdim), jnp.float32),
            jax.ShapeDtypeStruct((e, hdim), jnp.float32),
        ],
        interpret=_INTERP,
    )(pq_rows, ea, wd16, wet, b1, w2t, b2, cw1t, cb1, cw2)


def _node_body(h_ref, x_ref, m0_ref, t0_ref, m1_ref, t1_ref,
               w1at_ref, w1bt_ref, nb1_ref,
               w2t_ref, nb2_ref, g_ref, b_ref, ho_ref, xo_ref):
    m_i = m0_ref[...] + m1_ref[...]
    x_agg = (t0_ref[...] + t1_ref[...])[:, :3]
    xo_ref[...] = x_ref[...] + x_agg
    hblk = h_ref[...]
    hh = _silu(jnp.dot(hblk, w1at_ref[...], preferred_element_type=jnp.float32)
               + jnp.dot(m_i, w1bt_ref[...], preferred_element_type=jnp.float32)
               + nb1_ref[...])
    ho = hblk + jnp.dot(hh, w2t_ref[...], preferred_element_type=jnp.float32) + nb2_ref[...]
    mu = jnp.mean(ho, axis=1, keepdims=True)
    ctr = ho - mu
    var = jnp.mean(ctr * ctr, axis=1, keepdims=True)
    ho_ref[...] = ctr * lax.rsqrt(var + 1e-5) * g_ref[...] + b_ref[...]


def _node_update(h, x, m0, t0, m1, t1, w1at, w1bt, nb1, w2t, nb2, g, b):
    n, d = h.shape
    hdim = w1bt.shape[0]
    bn = 1000
    grid = (n // bn,)
    full = lambda i: (0, 0)
    return pl.pallas_call(
        _node_body,
        grid=grid,
        in_specs=[
            pl.BlockSpec((bn, d), lambda i: (i, 0)),
            pl.BlockSpec((bn, 3), lambda i: (i, 0)),
            pl.BlockSpec((bn, hdim), lambda i: (i, 0)),
            pl.BlockSpec((bn, hdim), lambda i: (i, 0)),
            pl.BlockSpec((bn, hdim), lambda i: (i, 0)),
            pl.BlockSpec((bn, hdim), lambda i: (i, 0)),
            pl.BlockSpec((d, hdim), full),
            pl.BlockSpec((hdim, hdim), full),
            pl.BlockSpec((1, hdim), full),
            pl.BlockSpec((hdim, d), full),
            pl.BlockSpec((1, d), full),
            pl.BlockSpec((1, d), full),
            pl.BlockSpec((1, d), full),
        ],
        out_specs=[
            pl.BlockSpec((bn, d), lambda i: (i, 0)),
            pl.BlockSpec((bn, 3), lambda i: (i, 0)),
        ],
        out_shape=[
            jax.ShapeDtypeStruct((n, d), jnp.float32),
            jax.ShapeDtypeStruct((n, 3), jnp.float32),
        ],
        interpret=_INTERP,
    )(h, x, m0, t0, m1, t1, w1at, w1bt, nb1, w2t, nb2, g, b)


def _pick_ch(ipe):
    for c in range(128, 7, -8):
        if ipe % c == 0:
            return c
    raise ValueError(ipe)


def _edge_gather(p, q, row, col):
    e = row.shape[0]
    w2 = p.shape[1]
    nw = 32
    ipe = e // nw
    ch = _pick_ch(ipe)
    ipw = ipe // ch
    hdim = w2 // 2
    mesh = plsc.VectorSubcoreMesh(core_axis_name="c", subcore_axis_name="s")

    @functools.partial(
        pl.kernel,
        out_type=jax.ShapeDtypeStruct((e, w2), jnp.float32),
        mesh=mesh,
        scratch_types=[
            pltpu.VMEM((ipe,), jnp.int32),
            pltpu.VMEM((ipe,), jnp.int32),
            pltpu.VMEM((ch, w2), jnp.float32),
            pltpu.VMEM((ch, w2), jnp.float32),
            pltpu.VMEM((ch, w2), jnp.float32),
            pltpu.VMEM((ch, w2), jnp.float32),
            pltpu.SemaphoreType.DMA,
            pltpu.SemaphoreType.DMA,
            pltpu.SemaphoreType.DMA,
            pltpu.SemaphoreType.DMA,
            pltpu.SemaphoreType.DMA,
            pltpu.SemaphoreType.DMA,
        ],
    )
    def gather_kernel(p_hbm, q_hbm, row_hbm, col_hbm, po,
                      ir_all, ic_all, pv0, qv0, pv1, qv1,
                      sp0, sq0, sp1, sq1, sw0, sw1):
        cid = lax.axis_index("c")
        sid = lax.axis_index("s")
        wid = sid * 2 + cid
        e0 = wid * ipe
        pltpu.sync_copy(row_hbm.at[pl.ds(e0, ipe)], ir_all)
        pltpu.sync_copy(col_hbm.at[pl.ds(e0, ipe)], ic_all)
        pvs = (pv0, pv1)
        qvs = (qv0, qv1)
        sps = (sp0, sp1)
        sqs = (sq0, sq1)
        sws = (sw0, sw1)

        def issue(j, s):
            lb = j * ch
            pltpu.async_copy(p_hbm.at[ir_all.at[pl.ds(lb, ch)]], pvs[s], sps[s])
            pltpu.async_copy(q_hbm.at[ic_all.at[pl.ds(lb, ch)]], qvs[s], sqs[s])

        issue(0, 0)

        def pair(i, carry):
            for b in (0, 1):
                j = 2 * i + b
                jn = j + 1

                @pl.when(jn < ipw)
                def _():
                    @pl.when(j >= 1)
                    def _():
                        pltpu.make_async_copy(
                            pvs[1 - b], po.at[pl.ds(e0 + (j - 1) * ch, ch)],
                            sws[1 - b]).wait()

                    issue(jn, 1 - b)

                @pl.when(j < ipw)
                def _():
                    lb = j * ch
                    pltpu.make_async_copy(
                        p_hbm.at[ir_all.at[pl.ds(lb, ch)]], pvs[b], sps[b]).wait()
                    pltpu.make_async_copy(
                        q_hbm.at[ic_all.at[pl.ds(lb, ch)]], qvs[b], sqs[b]).wait()
                    p_v = pvs[b]
                    q_v = qvs[b]

                    @plsc.parallel_loop(0, ch, step=1, unroll=4)
                    def crow(r):
                        for k in range(hdim // 16):
                            sl = pl.ds(k * 16, 16)
                            p_v[r, sl] = p_v[r, sl] + q_v[r, sl]
                        dsl = pl.ds(hdim, 16)
                        p_v[r, dsl] = p_v[r, dsl] - q_v[r, dsl]

                    pltpu.async_copy(p_v, po.at[pl.ds(e0 + lb, ch)], sws[b])

            return carry

        lax.fori_loop(0, (ipw + 1) // 2, pair, 0)
        for jl in (ipw - 1, ipw - 2):
            pltpu.make_async_copy(
                pvs[jl % 2], po.at[pl.ds(e0 + jl * ch, ch)], sws[jl % 2]).wait()

    return gather_kernel(p, q, row, col)


def _edge_scatter(mupd, tupd, row, zm):
    e, hdim = mupd.shape
    n = zm.shape[0]
    npc = 16
    ipe = e // npc
    ch = _pick_ch(ipe)
    while (n // 16) % ch:
        ch //= 2
    ipw = ipe // ch
    rows_per_tile = n // 16
    stage_steps = rows_per_tile // ch
    mesh = plsc.VectorSubcoreMesh(core_axis_name="c", subcore_axis_name="s")

    @functools.partial(
        pl.kernel,
        out_type=[
            jax.ShapeDtypeStruct((n, hdim), jnp.float32),
            jax.ShapeDtypeStruct((n, hdim), jnp.float32),
        ],
        mesh=mesh,
        scratch_types=[
            pltpu.VMEM((ch,), jnp.int32),
            pltpu.VMEM((ch,), jnp.int32),
            pltpu.VMEM((ch, hdim), jnp.float32),
            pltpu.VMEM((ch, hdim), jnp.float32),
            pltpu.VMEM_SHARED((n, hdim), jnp.float32),
            pltpu.SemaphoreType.DMA,
            pltpu.SemaphoreType.DMA,
            pltpu.SemaphoreType.DMA,
            pltpu.SemaphoreType.DMA,
        ],
    )
    def scatter_kernel(m_hbm, t_hbm, row_hbm, zm_hbm, macc, tacc,
                       iv0, iv1, uv0, uv1, acc_sh, su0, su1, si0, si1):
        cid = lax.axis_index("c")
        sid = lax.axis_index("s")
        r0 = sid * rows_per_tile
        e0 = sid * ipe
        ivs = (iv0, iv1)
        uvs = (uv0, uv1)
        sus = (su0, su1)
        sis = (si0, si1)

        def zstep(j, carry):
            rb = r0 + j * ch
            pltpu.sync_copy(zm_hbm.at[pl.ds(rb, ch)], uv0)
            pltpu.sync_copy(uv0, acc_sh.at[pl.ds(rb, ch)])
            return carry

        lax.fori_loop(0, stage_steps, zstep, 0)
        plsc.subcore_barrier()

        def make_loop(src_hbm):
            def issue(j, s):
                base = e0 + j * ch
                pltpu.async_copy(row_hbm.at[pl.ds(base, ch)], ivs[s], sis[s])
                pltpu.async_copy(src_hbm.at[pl.ds(base, ch)], uvs[s], sus[s])

            issue(0, 0)

            def pair(i, carry):
                for b in (0, 1):
                    j = 2 * i + b
                    jn = j + 1

                    @pl.when(jn < ipw)
                    def _():
                        issue(jn, 1 - b)

                    @pl.when(j < ipw)
                    def _():
                        base = e0 + j * ch
                        pltpu.make_async_copy(
                            row_hbm.at[pl.ds(base, ch)], ivs[b], sis[b]).wait()
                        pltpu.make_async_copy(
                            src_hbm.at[pl.ds(base, ch)], uvs[b], sus[b]).wait()
                        pltpu.sync_copy(uvs[b], acc_sh.at[ivs[b]], add=True)

                return carry

            lax.fori_loop(0, (ipw + 1) // 2, pair, 0)

        @pl.when(cid == 0)
        def _():
            make_loop(m_hbm)

        @pl.when(cid == 1)
        def _():
            make_loop(t_hbm)

        plsc.subcore_barrier()

        def dstep(j, carry):
            rb = r0 + j * ch
            pltpu.sync_copy(acc_sh.at[pl.ds(rb, ch)], uv0)

            @pl.when(cid == 0)
            def _():
                pltpu.sync_copy(uv0, macc.at[pl.ds(rb, ch)])

            @pl.when(cid == 1)
            def _():
                pltpu.sync_copy(uv0, tacc.at[pl.ds(rb, ch)])

            return carry

        lax.fori_loop(0, stage_steps, dstep, 0)

    return scatter_kernel(mupd, tupd, row, zm)


def kernel(h, x, edge_index, edge_attr, msg_w1, msg_b1, msg_w2, msg_b2,
           node_w1, node_b1, node_w2, node_b2, coord_w1, coord_b1, coord_w2,
           ln_g, ln_b):
    n, d = h.shape
    e = edge_index.shape[1]
    hdim = msg_w2.shape[0]
    de = edge_attr.shape[1]

    row = edge_index[0]
    col = edge_index[1]

    wat = msg_w1[:, :d].T
    wbt = msg_w1[:, d:2 * d].T
    wd = msg_w1[:, 2 * d]
    wet = msg_w1[:, 2 * d + 1:].T
    b1_eff = (msg_b1 + 1e-8 * wd)[None, :]
    wd16 = jnp.broadcast_to(wd[None, :], (XPAD, hdim))
    w2t = msg_w2.T
    b2 = msg_b2[None, :]
    cw1t = coord_w1.T
    cb1 = coord_b1[None, :]
    cw2 = coord_w2
    w1at = node_w1[:, :d].T
    w1bt = node_w1[:, d:].T
    nb1 = node_b1[None, :]
    nw2t = node_w2.T
    nb2 = node_b2[None, :]
    g = ln_g[None, :]
    b = ln_b[None, :]

    p, q = _node_proj(h, x, wat, wbt)

    n_pad = -(-n // 2048) * 2048
    zm = jnp.zeros((n_pad, hdim), dtype=jnp.float32)
    nslice = 2
    es = e // nslice
    parts = []
    for k in range(nslice):
        sl = slice(k * es, (k + 1) * es)
        pq_rows = _edge_gather(p, q, row[sl], col[sl])
        mout, tout = _edge_mlp(pq_rows, edge_attr[sl],
                               wd16, wet, b1_eff, w2t, b2, cw1t, cb1, cw2)
        parts.extend(_edge_scatter(mout, tout, row[sl], zm))

    h_out, x_out = _node_update(h, x, *parts,
                                w1at, w1bt, nb1, nw2t, nb2, g, b)
    return (h_out, x_out)
